# baseline (speedup 1.0000x reference)
import functools

import jax
import jax.numpy as jnp
from jax import lax
from jax.experimental import pallas as pl
from jax.experimental.pallas import tpu as pltpu
from jax.experimental.pallas import tpu_sc as plsc

NC = 2
NS = 16
NW = NC * NS

B = 16384
S = 20
D = 64
VT = 2001
VTITLE = 1001
PW = D // 2
RPW = B // NW
CB = 32
NCHUNK = RPW // CB
import numpy as np

MASK_HI = np.int32(-65536)


def _sc_body(title_ids_hbm, tok_hbm, title_tab_hbm, text_tab_hbm,
             out_hbm, textv, titlev, tokv, tidv, outv, sem):
    wid = lax.axis_index("s") * NC + lax.axis_index("c")

    pltpu.sync_copy(text_tab_hbm, textv)
    pltpu.sync_copy(title_tab_hbm, titlev)
    pltpu.sync_copy(tok_hbm.at[wid], tokv)
    pltpu.sync_copy(title_ids_hbm.at[pl.ds(wid * RPW, RPW)], tidv)

    def remap_tok(i, _):
        t = tokv[pl.ds(i * 16, 16)]
        tokv[pl.ds(i * 16, 16)] = jnp.where(t == 0, jnp.int32(VT - 1), t) * PW
        return _

    lax.fori_loop(0, S * RPW // 16, remap_tok, None)

    def scale_tid(i, _):
        tidv[pl.ds(i * 16, 16)] = tidv[pl.ds(i * 16, 16)] * PW
        return _

    lax.fori_loop(0, RPW // 16, scale_tid, None)

    iota = lax.iota(jnp.int32, 16)

    def group_body(r0, rowvec):
        a = [tokv[pl.ds(s * RPW + r0, 16)] for s in range(S)]
        ta = tidv[pl.ds(r0, 16)]
        for c in range(PW):
            acc = plsc.bitcast(plsc.load_gather(textv, [a[0]]), jnp.bfloat16)
            a[0] = a[0] + 1
            for s in range(1, S):
                v = plsc.load_gather(textv, [a[s]])
                a[s] = a[s] + 1
                acc = jnp.maximum(acc, plsc.bitcast(v, jnp.bfloat16))
            ai = plsc.bitcast(acc, jnp.int32)
            lo = plsc.bitcast(jnp.left_shift(ai, 16), jnp.float32)
            hi = plsc.bitcast(jnp.bitwise_and(ai, MASK_HI), jnp.float32)
            plsc.store_scatter(outv, [rowvec + (D + 2 * c)], lo)
            plsc.store_scatter(outv, [rowvec + (D + 2 * c + 1)], hi)
            tv = plsc.load_gather(titlev, [ta])
            ta = ta + 1
            tlo = plsc.bitcast(jnp.left_shift(tv, 16), jnp.float32)
            thi = plsc.bitcast(jnp.bitwise_and(tv, MASK_HI), jnp.float32)
            plsc.store_scatter(outv, [rowvec + 2 * c], tlo)
            plsc.store_scatter(outv, [rowvec + (2 * c + 1)], thi)

    def chunk_body(g, _):
        def grp_body(grp, __):
            group_body(g * CB + grp * 16, (iota + grp * 16) * (2 * D))
            return __

        lax.fori_loop(0, CB // 16, grp_body, None)
        pltpu.sync_copy(
            outv, out_hbm.at[pl.ds((wid * RPW + g * CB) * 2 * D, CB * 2 * D)])
        return _

    lax.fori_loop(0, NCHUNK, chunk_body, None)


@jax.jit
def _run(title_ids, tok_bw, title_pack, text_pack):
    mesh = plsc.VectorSubcoreMesh(core_axis_name="c", subcore_axis_name="s")
    f = functools.partial(
        pl.kernel,
        out_type=jax.ShapeDtypeStruct((B * 2 * D,), jnp.float32),
        mesh=mesh,
        compiler_params=pltpu.CompilerParams(needs_layout_passes=False),
        scratch_types=[
            pltpu.VMEM((VT * PW,), jnp.int32),
            pltpu.VMEM((VTITLE * PW,), jnp.int32),
            pltpu.VMEM((S * RPW,), jnp.int32),
            pltpu.VMEM((RPW,), jnp.int32),
            pltpu.VMEM((CB * 2 * D,), jnp.float32),
            pltpu.SemaphoreType.DMA,
        ],
    )(_sc_body)
    return f(title_ids, tok_bw, title_pack, text_pack)


def kernel(title_ids, token_ids, title_table, text_table):
    text_aug = jnp.concatenate(
        [text_table, jnp.full((1, D), -1e9, jnp.float32)], axis=0)
    text_pack = lax.bitcast_convert_type(
        text_aug.astype(jnp.bfloat16).reshape(VT, PW, 2), jnp.int32
    ).reshape(-1)
    title_pack = lax.bitcast_convert_type(
        title_table.astype(jnp.bfloat16).reshape(VTITLE, PW, 2), jnp.int32
    ).reshape(-1)
    tok_bw = token_ids.reshape(NW, RPW, S).transpose(0, 2, 1).reshape(NW, S * RPW)
    out = _run(title_ids, tok_bw, title_pack, text_pack)
    return out.reshape(B, 2 * D)


# speedup vs baseline: 2.8034x; 2.8034x over previous
import functools

import jax
import jax.numpy as jnp
import numpy as np
from jax import lax
from jax.experimental import pallas as pl
from jax.experimental.pallas import tpu as pltpu
from jax.experimental.pallas import tpu_sc as plsc

NC = 2
NS = 16
NW = NC * NS

B = 16384
S = 20
D = 64
VT = 2001
VTITLE = 1001
RPW = B // NW
CB = 32
NCHUNK = RPW // CB
SP = 32
PW = D // 2
MASK_HI = np.int32(-65536)


def _treemax(vals):
    while len(vals) > 1:
        nxt = [jnp.maximum(a, b) for a, b in zip(vals[::2], vals[1::2])]
        if len(vals) % 2:
            nxt.append(vals[-1])
        vals = nxt
    return vals[0]


def _unpack_store(ai, ref, base):
    ref[pl.ds(base, 16)] = plsc.bitcast(jnp.left_shift(ai, 16), jnp.float32)
    ref[pl.ds(base + 16, 16)] = plsc.bitcast(
        jnp.bitwise_and(ai, MASK_HI), jnp.float32)


def _sc_body(title_ids_hbm, tok_hbm, title_tab_hbm, text_tab_hbm,
             out_hbm, textv, titlev, tokv, tidv, outv, sem):
    wid = lax.axis_index("s") * NC + lax.axis_index("c")

    pltpu.sync_copy(text_tab_hbm, textv)
    pltpu.sync_copy(title_tab_hbm, titlev)
    pltpu.sync_copy(tok_hbm.at[pl.ds(wid * (SP * RPW), SP * RPW)], tokv)
    pltpu.sync_copy(title_ids_hbm.at[pl.ds(wid * RPW, RPW)], tidv)

    def remap_tok(i, _):
        t = tokv[pl.ds(i * 16, 16)]
        tokv[pl.ds(i * 16, 16)] = jnp.where(t == 0, jnp.int32(VT - 1), t) * PW
        return _

    lax.fori_loop(0, SP * RPW // 16, remap_tok, None)

    def scale_tid(i, _):
        tidv[pl.ds(i * 16, 16)] = tidv[pl.ds(i * 16, 16)] * PW
        return _

    lax.fori_loop(0, RPW // 16, scale_tid, None)

    def chunk_body(g, _):
        def grp_body(gg, __):
            row0 = g * CB + gg * 16
            tvec = tidv[pl.ds(row0, 16)]
            for i in range(16):
                rs = (row0 + i) * SP
                t0 = tokv[pl.ds(rs, 16)]
                t1 = tokv[pl.ds(rs + 16, 16)]
                offs = [t0[s] for s in range(16)] + [t1[s] for s in range(S - 16)]
                ob = (gg * 16 + i) * 2 * D
                ta = tvec[i]
                for h in (0, 1):
                    vals = [
                        plsc.bitcast(textv[pl.ds(offs[s] + h * 16, 16)],
                                     jnp.bfloat16)
                        for s in range(S)
                    ]
                    acc = plsc.bitcast(_treemax(vals), jnp.int32)
                    _unpack_store(acc, outv, ob + D + h * 32)
                    _unpack_store(titlev[pl.ds(ta + h * 16, 16)], outv,
                                  ob + h * 32)
            return __

        lax.fori_loop(0, CB // 16, grp_body, None)
        pltpu.sync_copy(
            outv, out_hbm.at[pl.ds((wid * RPW + g * CB) * 2 * D, CB * 2 * D)])
        return _

    lax.fori_loop(0, NCHUNK, chunk_body, None)


def _permute(table_f32):
    v = table_f32.shape[0]
    t = table_f32.astype(jnp.bfloat16).reshape(v, 2, 2, 16)
    perm = t.transpose(0, 1, 3, 2).reshape(v, PW, 2)
    return lax.bitcast_convert_type(perm, jnp.int32).reshape(-1)


@jax.jit
def _run(title_ids, tok_bw, title_perm, text_perm):
    mesh = plsc.VectorSubcoreMesh(core_axis_name="c", subcore_axis_name="s")
    f = functools.partial(
        pl.kernel,
        out_type=jax.ShapeDtypeStruct((B * 2 * D,), jnp.float32),
        mesh=mesh,
        compiler_params=pltpu.CompilerParams(needs_layout_passes=False),
        scratch_types=[
            pltpu.VMEM((VT * PW,), jnp.int32),
            pltpu.VMEM((VTITLE * PW,), jnp.int32),
            pltpu.VMEM((SP * RPW,), jnp.int32),
            pltpu.VMEM((RPW,), jnp.int32),
            pltpu.VMEM((CB * 2 * D,), jnp.float32),
            pltpu.SemaphoreType.DMA,
        ],
    )(_sc_body)
    return f(title_ids, tok_bw, title_perm, text_perm)


def kernel(title_ids, token_ids, title_table, text_table):
    text_aug = jnp.concatenate(
        [text_table, jnp.full((1, D), -1e9, jnp.float32)], axis=0)
    text_perm = _permute(text_aug)
    title_perm = _permute(title_table)
    tok_bw = jnp.pad(
        token_ids.reshape(NW, RPW, S), ((0, 0), (0, 0), (0, SP - S))
    ).reshape(-1)
    out = _run(title_ids, tok_bw, title_perm, text_perm)
    return out.reshape(B, 2 * D)


# speedup vs baseline: 2.9899x; 1.0665x over previous
import functools

import jax
import jax.numpy as jnp
import numpy as np
from jax import lax
from jax.experimental import pallas as pl
from jax.experimental.pallas import tpu as pltpu
from jax.experimental.pallas import tpu_sc as plsc

NC = 2
NS = 16
NW = NC * NS

B = 16384
S = 20
D = 64
VT = 2001
VTITLE = 1001
RPW = B // NW
CB = 32
NCHUNK = RPW // CB
SP = 32
PW = D // 2
MASK_HI = np.int32(-65536)


def _treemax(vals):
    while len(vals) > 1:
        nxt = [jnp.maximum(a, b) for a, b in zip(vals[::2], vals[1::2])]
        if len(vals) % 2:
            nxt.append(vals[-1])
        vals = nxt
    return vals[0]


def _unpack_store(ai, ref, base):
    ref[pl.ds(base, 16)] = plsc.bitcast(jnp.left_shift(ai, 16), jnp.float32)
    ref[pl.ds(base + 16, 16)] = plsc.bitcast(
        jnp.bitwise_and(ai, MASK_HI), jnp.float32)


def _sc_body(title_ids_hbm, tok_hbm, title_tab_hbm, text_tab_hbm,
             out_hbm, textv, titlev, tokv, tidv, outv, sem, sem_tab):
    wid = lax.axis_index("s") * NC + lax.axis_index("c")

    pltpu.sync_copy(tok_hbm.at[pl.ds(wid * (SP * RPW), SP * RPW)], tokv)
    pltpu.sync_copy(title_ids_hbm.at[pl.ds(wid * RPW, RPW)], tidv)
    text_dma = pltpu.async_copy(text_tab_hbm, textv, sem_tab)
    title_dma = pltpu.async_copy(title_tab_hbm, titlev, sem_tab)

    def remap_tok(i, _):
        t = tokv[pl.ds(i * 16, 16)]
        tokv[pl.ds(i * 16, 16)] = jnp.where(t == 0, jnp.int32(VT - 1), t) * PW
        return _

    lax.fori_loop(0, SP * RPW // 16, remap_tok, None)

    def scale_tid(i, _):
        tidv[pl.ds(i * 16, 16)] = tidv[pl.ds(i * 16, 16)] * PW
        return _

    lax.fori_loop(0, RPW // 16, scale_tid, None)

    text_dma.wait()
    title_dma.wait()

    def chunk_body(g, _):
        ob0 = lax.rem(g, 2) * (CB * 2 * D)

        @pl.when(g >= 2)
        def _wait_prev():
            pltpu.make_async_copy(
                outv.at[pl.ds(0, CB * 2 * D)], out_hbm.at[pl.ds(0, CB * 2 * D)], sem).wait()

        def grp_body(gg, __):
            row0 = g * CB + gg * 16
            tvec = tidv[pl.ds(row0, 16)]
            for i in range(16):
                rs = (row0 + i) * SP
                t0 = tokv[pl.ds(rs, 16)]
                t1 = tokv[pl.ds(rs + 16, 16)]
                offs = [t0[s] for s in range(16)] + [t1[s] for s in range(S - 16)]
                ob = (gg * 16 + i) * 2 * D
                ta = tvec[i]
                for h in (0, 1):
                    vals = [
                        plsc.bitcast(textv[pl.ds(offs[s] + h * 16, 16)],
                                     jnp.bfloat16)
                        for s in range(S)
                    ]
                    acc = plsc.bitcast(_treemax(vals), jnp.int32)
                    _unpack_store(acc, outv, ob0 + ob + D + h * 32)
                    _unpack_store(titlev[pl.ds(ta + h * 16, 16)], outv,
                                  ob0 + ob + h * 32)
            return __

        lax.fori_loop(0, CB // 16, grp_body, None)
        pltpu.async_copy(
            outv.at[pl.ds(ob0, CB * 2 * D)],
            out_hbm.at[pl.ds((wid * RPW + g * CB) * 2 * D, CB * 2 * D)], sem)
        return _

    lax.fori_loop(0, NCHUNK, chunk_body, None)
    for _ in range(2):
        pltpu.make_async_copy(
            outv.at[pl.ds(0, CB * 2 * D)], out_hbm.at[pl.ds(0, CB * 2 * D)], sem).wait()


def _permute(table_f32):
    v = table_f32.shape[0]
    t = table_f32.astype(jnp.bfloat16).reshape(v, 2, 2, 16)
    perm = t.transpose(0, 1, 3, 2).reshape(v, PW, 2)
    return lax.bitcast_convert_type(perm, jnp.int32).reshape(-1)


@jax.jit
def _run(title_ids, tok_bw, title_perm, text_perm):
    mesh = plsc.VectorSubcoreMesh(core_axis_name="c", subcore_axis_name="s")
    f = functools.partial(
        pl.kernel,
        out_type=jax.ShapeDtypeStruct((B * 2 * D,), jnp.float32),
        mesh=mesh,
        compiler_params=pltpu.CompilerParams(needs_layout_passes=False),
        scratch_types=[
            pltpu.VMEM((VT * PW,), jnp.int32),
            pltpu.VMEM((VTITLE * PW,), jnp.int32),
            pltpu.VMEM((SP * RPW,), jnp.int32),
            pltpu.VMEM((RPW,), jnp.int32),
            pltpu.VMEM((2 * CB * 2 * D,), jnp.float32),
            pltpu.SemaphoreType.DMA,
            pltpu.SemaphoreType.DMA,
        ],
    )(_sc_body)
    return f(title_ids, tok_bw, title_perm, text_perm)


def kernel(title_ids, token_ids, title_table, text_table):
    text_aug = jnp.concatenate(
        [text_table, jnp.full((1, D), -1e9, jnp.float32)], axis=0)
    text_perm = _permute(text_aug)
    title_perm = _permute(title_table)
    tok_bw = jnp.pad(
        token_ids.reshape(NW, RPW, S), ((0, 0), (0, 0), (0, SP - S))
    ).reshape(-1)
    out = _run(title_ids, tok_bw, title_perm, text_perm)
    return out.reshape(B, 2 * D)


# speedup vs baseline: 3.3605x; 1.1240x over previous
import functools

import jax
import jax.numpy as jnp
import numpy as np
from jax import lax
from jax.experimental import pallas as pl
from jax.experimental.pallas import tpu as pltpu
from jax.experimental.pallas import tpu_sc as plsc

NC = 2
NS = 16
NW = NC * NS

B = 16384
S = 20
D = 64
VT = 2001
VTITLE = 1001
RPW = B // NW
CB = 32
NCHUNK = RPW // CB
PW = D // 2
TPW = S // 2
OTILE = CB * 2 * D
MASK_HI = np.int32(-65536)


def _treemax(vals):
    while len(vals) > 1:
        nxt = [jnp.maximum(a, b) for a, b in zip(vals[::2], vals[1::2])]
        if len(vals) % 2:
            nxt.append(vals[-1])
        vals = nxt
    return vals[0]


def _unpack_store(ai, ref, base):
    ref[pl.ds(base, 16)] = plsc.bitcast(jnp.left_shift(ai, 16), jnp.float32)
    ref[pl.ds(base + 16, 16)] = plsc.bitcast(
        jnp.bitwise_and(ai, MASK_HI), jnp.float32)


def _sc_body(title_ids_hbm, tok_hbm, title_tab_hbm, text_tab_hbm,
             out_hbm, textv, titlev, evv, odv, pkv, tidv, outv, sem, sem_tab):
    wid = lax.axis_index("s") * NC + lax.axis_index("c")

    pltpu.sync_copy(tok_hbm.at[pl.ds(wid * (2 * RPW * TPW), RPW * TPW)], evv)
    pltpu.sync_copy(
        tok_hbm.at[pl.ds(wid * (2 * RPW * TPW) + RPW * TPW, RPW * TPW)], odv)
    pltpu.sync_copy(title_ids_hbm.at[pl.ds(wid * RPW, RPW)], tidv)
    text_dma = pltpu.async_copy(text_tab_hbm, textv, sem_tab)
    title_dma = pltpu.async_copy(title_tab_hbm, titlev, sem_tab)

    def remap_pack(i, _):
        e = evv[pl.ds(i * 16, 16)]
        o = odv[pl.ds(i * 16, 16)]
        e = jnp.where(e == 0, jnp.int32(VT - 1), e) * PW
        o = jnp.where(o == 0, jnp.int32(VT - 1), o) * PW
        pkv[pl.ds(i * 16, 16)] = jnp.bitwise_or(e, jnp.left_shift(o, 16))
        return _

    lax.fori_loop(0, RPW * TPW // 16, remap_pack, None)

    def scale_tid(i, _):
        tidv[pl.ds(i * 16, 16)] = tidv[pl.ds(i * 16, 16)] * PW
        return _

    lax.fori_loop(0, RPW // 16, scale_tid, None)

    text_dma.wait()
    title_dma.wait()

    def chunk_body(g, _):
        ob0 = lax.rem(g, 2) * OTILE

        @pl.when(g >= 2)
        def _wait_prev():
            pltpu.make_async_copy(
                outv.at[pl.ds(0, OTILE)], out_hbm.at[pl.ds(0, OTILE)],
                sem).wait()

        def grp_body(gg, __):
            row0 = g * CB + gg * 16
            tvec = tidv[pl.ds(row0, 16)]
            pv = [pkv[pl.ds(row0 * TPW + j * 16, 16)] for j in range(TPW)]
            for i in range(16):
                offs = []
                for k in range(TPW):
                    p = i * TPW + k
                    w = pv[p // 16][p % 16]
                    offs.append(jnp.bitwise_and(w, jnp.int32(0xFFFF)))
                    offs.append(lax.shift_right_logical(w, 16))
                ob = ob0 + (gg * 16 + i) * 2 * D
                ta = tvec[i]
                for h in (0, 1):
                    vals = [
                        plsc.bitcast(textv[pl.ds(offs[s] + h * 16, 16)],
                                     jnp.bfloat16)
                        for s in range(S)
                    ]
                    acc = plsc.bitcast(_treemax(vals), jnp.int32)
                    _unpack_store(acc, outv, ob + D + h * 32)
                    _unpack_store(titlev[pl.ds(ta + h * 16, 16)], outv,
                                  ob + h * 32)
            return __

        lax.fori_loop(0, CB // 16, grp_body, None)
        pltpu.async_copy(
            outv.at[pl.ds(ob0, OTILE)],
            out_hbm.at[pl.ds((wid * RPW + g * CB) * 2 * D, OTILE)], sem)
        return _

    lax.fori_loop(0, NCHUNK, chunk_body, None)
    for _ in range(2):
        pltpu.make_async_copy(
            outv.at[pl.ds(0, OTILE)], out_hbm.at[pl.ds(0, OTILE)], sem).wait()


def _permute(table_f32):
    v = table_f32.shape[0]
    t = table_f32.astype(jnp.bfloat16).reshape(v, 2, 2, 16)
    perm = t.transpose(0, 1, 3, 2).reshape(v, PW, 2)
    return lax.bitcast_convert_type(perm, jnp.int32).reshape(-1)


@jax.jit
def _run(title_ids, tok_pk, title_perm, text_perm):
    mesh = plsc.VectorSubcoreMesh(core_axis_name="c", subcore_axis_name="s")
    f = functools.partial(
        pl.kernel,
        out_type=jax.ShapeDtypeStruct((B * 2 * D,), jnp.float32),
        mesh=mesh,
        compiler_params=pltpu.CompilerParams(
            needs_layout_passes=False, disable_bounds_checks=True),
        scratch_types=[
            pltpu.VMEM((VT * PW,), jnp.int32),
            pltpu.VMEM((VTITLE * PW,), jnp.int32),
            pltpu.VMEM((RPW * TPW,), jnp.int32),
            pltpu.VMEM((RPW * TPW,), jnp.int32),
            pltpu.VMEM((RPW * TPW,), jnp.int32),
            pltpu.VMEM((RPW,), jnp.int32),
            pltpu.VMEM((2 * OTILE,), jnp.float32),
            pltpu.SemaphoreType.DMA,
            pltpu.SemaphoreType.DMA,
        ],
    )(_sc_body)
    return f(title_ids, tok_pk, title_perm, text_perm)


def kernel(title_ids, token_ids, title_table, text_table):
    text_aug = jnp.concatenate(
        [text_table, jnp.full((1, D), -1e9, jnp.float32)], axis=0)
    text_perm = _permute(text_aug)
    title_perm = _permute(title_table)
    tks = token_ids.reshape(NW, RPW, S)
    tok_pk = jnp.stack(
        [tks[:, :, 0::2].reshape(NW, -1), tks[:, :, 1::2].reshape(NW, -1)],
        axis=1).reshape(-1)
    out = _run(title_ids, tok_pk, title_perm, text_perm)
    return out.reshape(B, 2 * D)
